# Initial kernel scaffold; baseline (speedup 1.0000x reference)
import functools

import jax
import jax.numpy as jnp
from jax import lax
from jax.experimental import pallas as pl
from jax.experimental.pallas import tpu as pltpu
from jax.experimental.pallas import tpu_sc as plsc

POS_DIM = 16
H = 4
D = 32

NC = 2
NS = 16
NW = NC * NS
CHUNK = 80


def _proj_body(feat_ref, wh_ref, wa_ref, hc_ref, attn_ref):
    x = feat_ref[...]
    hc_ref[...] = jnp.dot(x, wh_ref[...], preferred_element_type=jnp.float32)
    attn_ref[...] = jnp.dot(x, wa_ref[...], preferred_element_type=jnp.float32)


def _node_proj(feat, W_h, W_a, block_rows):
    n = feat.shape[0]
    grid = n // block_rows
    return pl.pallas_call(
        _proj_body,
        grid=(grid,),
        in_specs=[
            pl.BlockSpec((block_rows, feat.shape[1]), lambda i: (i, 0)),
            pl.BlockSpec(W_h.shape, lambda i: (0, 0)),
            pl.BlockSpec(W_a.shape, lambda i: (0, 0)),
        ],
        out_specs=[
            pl.BlockSpec((block_rows, W_h.shape[1]), lambda i: (i, 0)),
            pl.BlockSpec((block_rows, W_a.shape[1]), lambda i: (i, 0)),
        ],
        out_shape=[
            jax.ShapeDtypeStruct((n, W_h.shape[1]), jnp.float32),
            jax.ShapeDtypeStruct((n, W_a.shape[1]), jnp.float32),
        ],
    )(feat, W_h, W_a)


def _edge_kernel(n_nodes, n_edges):
    ept = n_edges // NW
    n_chunks = ept // CHUNK
    rpt = n_nodes // NS
    mesh = plsc.VectorSubcoreMesh(core_axis_name="c", subcore_axis_name="s")

    @functools.partial(
        pl.kernel,
        mesh=mesh,
        out_type=[
            jax.ShapeDtypeStruct((NC, n_nodes, H * D), jnp.float32),
            jax.ShapeDtypeStruct((NC, n_nodes, 8), jnp.float32),
        ],
        scratch_types=[
            pltpu.VMEM((n_nodes * 8,), jnp.float32),
            pltpu.VMEM((CHUNK,), jnp.int32),
            pltpu.VMEM((CHUNK,), jnp.int32),
            pltpu.VMEM((CHUNK, H * D), jnp.float32),
            pltpu.VMEM((CHUNK, 8), jnp.float32),
            pltpu.VMEM_SHARED((n_nodes, H * D), jnp.float32),
            pltpu.VMEM_SHARED((n_nodes, 8), jnp.float32),
            pltpu.SemaphoreType.DMA,
        ],
    )
    def kern(attn_hbm, edges_hbm, hc_hbm, zmsg_hbm, zden_hbm,
             out_msg, out_den,
             att_v, src_v, dst_v, rows_v, ex_v, msg_sh, den_sh, sem):
        cid = lax.axis_index("c")
        sid = lax.axis_index("s")
        wid = cid * NS + sid

        pltpu.sync_copy(attn_hbm, att_v)
        pltpu.sync_copy(zmsg_hbm.at[pl.ds(sid * rpt, rpt)],
                        msg_sh.at[pl.ds(sid * rpt, rpt)])
        pltpu.sync_copy(zden_hbm.at[pl.ds(sid * rpt, rpt)],
                        den_sh.at[pl.ds(sid * rpt, rpt)])
        pltpu.sync_copy(zden_hbm.at[pl.ds(0, CHUNK)], ex_v)
        plsc.subcore_barrier()

        lanes = lax.iota(jnp.int32, 16)

        def group(g, carry):
            off = g * 16
            src16 = src_v[pl.ds(off, 16)]
            dst16 = dst_v[pl.ds(off, 16)]
            for h in range(H):
                s = plsc.load_gather(att_v, [src16 * 8 + h])
                d = plsc.load_gather(att_v, [dst16 * 8 + (4 + h)])
                x = s + d
                ex = jnp.exp(jnp.maximum(x, x * 0.2))
                plsc.store_scatter(
                    ex_v, [off + lanes, jnp.full((16,), h, jnp.int32)], ex)
            for e in range(16):
                row = off + e
                for h in range(H):
                    sc = ex_v[row, h]
                    vsc = jnp.full((16,), sc)
                    for j in range(2):
                        col = h * D + j * 16
                        rows_v[row, pl.ds(col, 16)] = (
                            rows_v[row, pl.ds(col, 16)] * vsc)
            return carry

        def chunk(i, carry):
            base = wid * ept + i * CHUNK
            pltpu.sync_copy(edges_hbm.at[0, pl.ds(base, CHUNK)], src_v)
            pltpu.sync_copy(edges_hbm.at[1, pl.ds(base, CHUNK)], dst_v)
            pltpu.async_copy(hc_hbm.at[src_v], rows_v, sem).wait()
            lax.fori_loop(0, CHUNK // 16, group, 0)
            pltpu.sync_copy(rows_v, msg_sh.at[dst_v], add=True)
            pltpu.sync_copy(ex_v, den_sh.at[dst_v], add=True)
            return carry

        lax.fori_loop(0, n_chunks, chunk, 0)
        plsc.subcore_barrier()

        pltpu.sync_copy(msg_sh.at[pl.ds(sid * rpt, rpt)],
                        out_msg.at[cid, pl.ds(sid * rpt, rpt)])
        pltpu.sync_copy(den_sh.at[pl.ds(sid * rpt, rpt)],
                        out_den.at[cid, pl.ds(sid * rpt, rpt)])

    return kern


def _finish_body(msg_ref, den_ref, feat_ref, out_ref):
    m = msg_ref[0] + msg_ref[1]
    dsum = den_ref[0] + den_ref[1]
    parts = []
    for h in range(H):
        parts.append(m[:, h * D:(h + 1) * D] / (dsum[:, h:h + 1] + 1e-9))
    out_ref[...] = feat_ref[...] + jnp.concatenate(parts, axis=1)


def _finish(msg, den, feat, block_rows):
    n = feat.shape[0]
    grid = n // block_rows
    return pl.pallas_call(
        _finish_body,
        grid=(grid,),
        in_specs=[
            pl.BlockSpec((NC, block_rows, H * D), lambda i: (0, i, 0)),
            pl.BlockSpec((NC, block_rows, 8), lambda i: (0, i, 0)),
            pl.BlockSpec((block_rows, H * D), lambda i: (i, 0)),
        ],
        out_specs=pl.BlockSpec((block_rows, H * D), lambda i: (i, 0)),
        out_shape=jax.ShapeDtypeStruct((n, H * D), jnp.float32),
    )(msg, den, feat)


def kernel(feat, edge_index, W_content, W_pos, attn_src, attn_dst,
           pos_attn_src, pos_attn_dst, att_comb):
    n, in_dim = feat.shape
    e = edge_index.shape[1]
    content_dim = in_dim - POS_DIM

    c0 = att_comb[:, 0]
    c1 = att_comb[:, 1]
    wc3 = W_content.reshape(content_dim, H, D)
    wp3 = W_pos.reshape(POS_DIM, H, D // 4)
    a_src_c = jnp.einsum("chd,hd->ch", wc3, attn_src[0]) * c0
    a_dst_c = jnp.einsum("chd,hd->ch", wc3, attn_dst[0]) * c0
    a_src_p = jnp.einsum("phk,hk->ph", wp3, pos_attn_src[0]) * c1
    a_dst_p = jnp.einsum("phk,hk->ph", wp3, pos_attn_dst[0]) * c1
    W_a = jnp.concatenate(
        [jnp.concatenate([a_src_c, a_src_p], axis=0),
         jnp.concatenate([a_dst_c, a_dst_p], axis=0)], axis=1)
    W_h = jnp.concatenate(
        [W_content, jnp.zeros((POS_DIM, H * D), jnp.float32)], axis=0)

    hc, attn8 = _node_proj(feat, W_h, W_a, block_rows=1000)

    zmsg = jnp.zeros((n, H * D), jnp.float32)
    zden = jnp.zeros((n, 8), jnp.float32)
    out_msg, out_den = _edge_kernel(n, e)(
        attn8.reshape(-1), edge_index, hc, zmsg, zden)

    return _finish(out_msg, out_den, feat, block_rows=1000)


# speedup vs baseline: 54.2313x; 54.2313x over previous
import functools

import jax
import jax.numpy as jnp
from jax import lax
from jax.experimental import pallas as pl
from jax.experimental.pallas import tpu as pltpu
from jax.experimental.pallas import tpu_sc as plsc

POS_DIM = 16
H = 4
D = 32
HD2 = H * D // 2

NC = 2
NS = 16
CHUNK = 80


def _proj_body(feat_ref, wh_ref, wa_ref, hc_ref, attn_ref):
    x = feat_ref[...]
    hc = jnp.dot(x, wh_ref[...], preferred_element_type=jnp.float32)
    hc_ref[0] = hc[:, :HD2]
    hc_ref[1] = hc[:, HD2:]
    attn_ref[...] = jnp.dot(x, wa_ref[...], preferred_element_type=jnp.float32)


def _node_proj(feat, W_h, W_a, block_rows):
    n = feat.shape[0]
    grid = n // block_rows
    return pl.pallas_call(
        _proj_body,
        grid=(grid,),
        in_specs=[
            pl.BlockSpec((block_rows, feat.shape[1]), lambda i: (i, 0)),
            pl.BlockSpec(W_h.shape, lambda i: (0, 0)),
            pl.BlockSpec(W_a.shape, lambda i: (0, 0)),
        ],
        out_specs=[
            pl.BlockSpec((NC, block_rows, HD2), lambda i: (0, i, 0)),
            pl.BlockSpec((block_rows, W_a.shape[1]), lambda i: (i, 0)),
        ],
        out_shape=[
            jax.ShapeDtypeStruct((NC, n, HD2), jnp.float32),
            jax.ShapeDtypeStruct((n, W_a.shape[1]), jnp.float32),
        ],
    )(feat, W_h, W_a)


def _edge_kernel(n_nodes, n_edges):
    ept = n_edges // NS
    n_chunks = ept // CHUNK
    rpt = n_nodes // NS
    mesh = plsc.VectorSubcoreMesh(core_axis_name="c", subcore_axis_name="s")

    @functools.partial(
        pl.kernel,
        mesh=mesh,
        compiler_params=pltpu.CompilerParams(
            needs_layout_passes=False, use_tc_tiling_on_sc=False),
        out_type=[
            jax.ShapeDtypeStruct((NC, NS, rpt, HD2), jnp.float32),
            jax.ShapeDtypeStruct((NC, NS, rpt, 8), jnp.float32),
        ],
        scratch_types=[
            pltpu.VMEM((n_nodes * 4,), jnp.float32),
            pltpu.VMEM((CHUNK,), jnp.int32),
            pltpu.VMEM((CHUNK,), jnp.int32),
            pltpu.VMEM((CHUNK,), jnp.int32),
            pltpu.VMEM((CHUNK, HD2), jnp.float32),
            pltpu.VMEM((CHUNK, 8), jnp.float32),
            pltpu.VMEM_SHARED((n_nodes, HD2), jnp.float32),
            pltpu.VMEM_SHARED((n_nodes, 8), jnp.float32),
            pltpu.SemaphoreType.DMA,
        ],
    )
    def kern(att0_hbm, att1_hbm, src_hbm, dst_hbm, hc_hbm, zmsg_hbm, zden_hbm,
             out_msg, out_den,
             att_v, src_v, dst_v, gid_v, rows_v, ex_v, msg_sh, den_sh, sem):
        cid = lax.axis_index("c")
        sid = lax.axis_index("s")

        @pl.when(cid == 0)
        def _():
            pltpu.sync_copy(att0_hbm, att_v)

        @pl.when(cid == 1)
        def _():
            pltpu.sync_copy(att1_hbm, att_v)

        pltpu.sync_copy(zmsg_hbm, msg_sh.at[pl.ds(sid * rpt, rpt)])
        pltpu.sync_copy(zden_hbm, den_sh.at[pl.ds(sid * rpt, rpt)])
        pltpu.sync_copy(zden_hbm.at[pl.ds(0, CHUNK)], ex_v)
        plsc.subcore_barrier()

        lanes = lax.iota(jnp.int32, 16)
        row_off = cid * n_nodes

        def chunk(i, carry):
            base = pl.multiple_of(sid * ept + i * CHUNK, 16)
            pltpu.sync_copy(src_hbm.at[pl.ds(base, CHUNK)], src_v)
            pltpu.sync_copy(dst_hbm.at[pl.ds(base, CHUNK)], dst_v)

            def logits(g, carry):
                off = g * 16
                src16 = src_v[pl.ds(off, 16)]
                dst16 = dst_v[pl.ds(off, 16)]
                gid_v[pl.ds(off, 16)] = src16 + row_off
                for hh in range(2):
                    s = plsc.load_gather(att_v, [src16 * 4 + hh])
                    d = plsc.load_gather(att_v, [dst16 * 4 + (2 + hh)])
                    x = s + d
                    ex = jnp.exp(jnp.maximum(x, x * 0.2))
                    plsc.store_scatter(
                        ex_v, [off + lanes, jnp.full((16,), hh, jnp.int32)],
                        ex)
                return carry

            lax.fori_loop(0, CHUNK // 16, logits, 0)
            pltpu.async_copy(hc_hbm.at[gid_v], rows_v, sem).wait()

            def scale(g, carry):
                off = g * 16
                ex0 = plsc.load_gather(
                    ex_v, [off + lanes, jnp.zeros((16,), jnp.int32)])
                ex1 = plsc.load_gather(
                    ex_v, [off + lanes, jnp.full((16,), 1, jnp.int32)])
                for e in range(16):
                    row = off + e
                    for hh, exv in ((0, ex0), (1, ex1)):
                        vsc = jnp.full((16,), exv[e])
                        for j in range(2):
                            col = hh * D + j * 16
                            rows_v[row, pl.ds(col, 16)] = (
                                rows_v[row, pl.ds(col, 16)] * vsc)
                return carry

            lax.fori_loop(0, CHUNK // 16, scale, 0)
            pltpu.sync_copy(rows_v, msg_sh.at[dst_v], add=True)
            pltpu.sync_copy(ex_v, den_sh.at[dst_v], add=True)
            return carry

        lax.fori_loop(0, n_chunks, chunk, 0)
        plsc.subcore_barrier()

        pltpu.sync_copy(msg_sh.at[pl.ds(sid * rpt, rpt)], out_msg.at[cid, sid])
        pltpu.sync_copy(den_sh.at[pl.ds(sid * rpt, rpt)], out_den.at[cid, sid])

    return kern


def _finish_body(msg_ref, den_ref, feat_ref, out_ref):
    parts = []
    for c in range(NC):
        for hh in range(2):
            m = msg_ref[c][:, hh * D:(hh + 1) * D]
            dn = den_ref[c][:, hh:hh + 1]
            parts.append(m / (dn + 1e-9))
    out_ref[...] = feat_ref[...] + jnp.concatenate(parts, axis=1)


def _finish(msg, den, feat, block_rows):
    n = feat.shape[0]
    grid = n // block_rows
    return pl.pallas_call(
        _finish_body,
        grid=(grid,),
        in_specs=[
            pl.BlockSpec((NC, block_rows, HD2), lambda i: (0, i, 0)),
            pl.BlockSpec((NC, block_rows, 8), lambda i: (0, i, 0)),
            pl.BlockSpec((block_rows, H * D), lambda i: (i, 0)),
        ],
        out_specs=pl.BlockSpec((block_rows, H * D), lambda i: (i, 0)),
        out_shape=jax.ShapeDtypeStruct((n, H * D), jnp.float32),
    )(msg, den, feat)


def kernel(feat, edge_index, W_content, W_pos, attn_src, attn_dst,
           pos_attn_src, pos_attn_dst, att_comb):
    n, in_dim = feat.shape
    e = edge_index.shape[1]
    content_dim = in_dim - POS_DIM

    c0 = att_comb[:, 0]
    c1 = att_comb[:, 1]
    wc3 = W_content.reshape(content_dim, H, D)
    wp3 = W_pos.reshape(POS_DIM, H, D // 4)
    a_src_c = jnp.einsum("chd,hd->ch", wc3, attn_src[0]) * c0
    a_dst_c = jnp.einsum("chd,hd->ch", wc3, attn_dst[0]) * c0
    a_src_p = jnp.einsum("phk,hk->ph", wp3, pos_attn_src[0]) * c1
    a_dst_p = jnp.einsum("phk,hk->ph", wp3, pos_attn_dst[0]) * c1
    W_a = jnp.concatenate(
        [jnp.concatenate([a_src_c, a_src_p], axis=0),
         jnp.concatenate([a_dst_c, a_dst_p], axis=0)], axis=1)
    W_h = jnp.concatenate(
        [W_content, jnp.zeros((POS_DIM, H * D), jnp.float32)], axis=0)

    hc2, attn8 = _node_proj(feat, W_h, W_a, block_rows=1000)

    att_tabs = []
    for c in range(NC):
        att_tabs.append(jnp.concatenate(
            [attn8[:, 2 * c:2 * c + 2], attn8[:, 4 + 2 * c:4 + 2 * c + 2]],
            axis=1).reshape(-1))

    zmsg = jnp.zeros((n // NS, HD2), jnp.float32)
    zden = jnp.zeros((n // NS, 8), jnp.float32)
    out_msg, out_den = _edge_kernel(n, e)(
        att_tabs[0], att_tabs[1], edge_index[0], edge_index[1],
        hc2.reshape(NC * n, HD2), zmsg, zden)

    return _finish(out_msg.reshape(NC, n, HD2),
                   out_den.reshape(NC, n, 8), feat, block_rows=1000)


# speedup vs baseline: 71.9114x; 1.3260x over previous
import functools

import jax
import jax.numpy as jnp
from jax import lax
from jax.experimental import pallas as pl
from jax.experimental.pallas import tpu as pltpu
from jax.experimental.pallas import tpu_sc as plsc

POS_DIM = 16
H = 4
D = 32
HD2 = H * D // 2

NC = 2
NS = 16
CHUNK = 80


def _proj_body(feat_ref, wh_ref, wa_ref, hc_ref, attn_ref):
    x = feat_ref[...]
    hc = jnp.dot(x, wh_ref[...], preferred_element_type=jnp.float32)
    hc_ref[0] = hc[:, :HD2]
    hc_ref[1] = hc[:, HD2:]
    attn_ref[...] = jnp.dot(x, wa_ref[...], preferred_element_type=jnp.float32)


def _node_proj(feat, W_h, W_a, block_rows):
    n = feat.shape[0]
    grid = n // block_rows
    return pl.pallas_call(
        _proj_body,
        grid=(grid,),
        in_specs=[
            pl.BlockSpec((block_rows, feat.shape[1]), lambda i: (i, 0)),
            pl.BlockSpec(W_h.shape, lambda i: (0, 0)),
            pl.BlockSpec(W_a.shape, lambda i: (0, 0)),
        ],
        out_specs=[
            pl.BlockSpec((NC, block_rows, HD2), lambda i: (0, i, 0)),
            pl.BlockSpec((block_rows, W_a.shape[1]), lambda i: (i, 0)),
        ],
        out_shape=[
            jax.ShapeDtypeStruct((NC, n, HD2), jnp.float32),
            jax.ShapeDtypeStruct((n, W_a.shape[1]), jnp.float32),
        ],
    )(feat, W_h, W_a)


def _edge_kernel(n_nodes, n_edges):
    ept = n_edges // NS
    n_chunks = ept // CHUNK
    rpt = n_nodes // NS
    mesh = plsc.VectorSubcoreMesh(core_axis_name="c", subcore_axis_name="s")

    @functools.partial(
        pl.kernel,
        mesh=mesh,
        compiler_params=pltpu.CompilerParams(
            needs_layout_passes=False, use_tc_tiling_on_sc=False),
        out_type=[
            jax.ShapeDtypeStruct((NC, NS, rpt, HD2), jnp.float32),
            jax.ShapeDtypeStruct((NC, NS, rpt, 8), jnp.float32),
        ],
        scratch_types=[
            pltpu.VMEM((n_nodes * 4,), jnp.float32),
            pltpu.VMEM((CHUNK,), jnp.int32),
            pltpu.VMEM((CHUNK,), jnp.int32),
            pltpu.VMEM((CHUNK,), jnp.int32),
            pltpu.VMEM((CHUNK,), jnp.int32),
            pltpu.VMEM((CHUNK,), jnp.int32),
            pltpu.VMEM((CHUNK,), jnp.int32),
            pltpu.VMEM((CHUNK, HD2), jnp.float32),
            pltpu.VMEM((CHUNK, HD2), jnp.float32),
            pltpu.VMEM((CHUNK, 8), jnp.float32),
            pltpu.VMEM((CHUNK, 8), jnp.float32),
            pltpu.VMEM_SHARED((n_nodes, HD2), jnp.float32),
            pltpu.VMEM_SHARED((n_nodes, 8), jnp.float32),
            pltpu.SemaphoreType.DMA,
            pltpu.SemaphoreType.DMA,
        ],
    )
    def kern(att0_hbm, att1_hbm, src_hbm, dst_hbm, hc_hbm, zmsg_hbm, zden_hbm,
             out_msg, out_den,
             att_v, src_v0, src_v1, dst_v0, dst_v1, gid_v0, gid_v1,
             rows_v0, rows_v1, ex_v0, ex_v1, msg_sh, den_sh, sem0, sem1):
        cid = lax.axis_index("c")
        sid = lax.axis_index("s")
        bufs = ((src_v0, dst_v0, gid_v0, rows_v0, ex_v0, sem0),
                (src_v1, dst_v1, gid_v1, rows_v1, ex_v1, sem1))

        @pl.when(cid == 0)
        def _():
            pltpu.sync_copy(att0_hbm, att_v)

        @pl.when(cid == 1)
        def _():
            pltpu.sync_copy(att1_hbm, att_v)

        pltpu.sync_copy(zmsg_hbm, msg_sh.at[pl.ds(sid * rpt, rpt)])
        pltpu.sync_copy(zden_hbm, den_sh.at[pl.ds(sid * rpt, rpt)])
        pltpu.sync_copy(zden_hbm.at[pl.ds(0, CHUNK)], ex_v0)
        pltpu.sync_copy(zden_hbm.at[pl.ds(0, CHUNK)], ex_v1)
        plsc.subcore_barrier()

        lanes = lax.iota(jnp.int32, 16)
        row_off = cid * n_nodes

        def start_chunk(ci, b):
            src_v, dst_v, gid_v, rows_v, ex_v, sem = b
            base = pl.multiple_of(sid * ept + ci * CHUNK, 16)
            pltpu.sync_copy(src_hbm.at[pl.ds(base, CHUNK)], src_v)
            pltpu.sync_copy(dst_hbm.at[pl.ds(base, CHUNK)], dst_v)

            def logits(g, carry):
                off = g * 16
                src16 = src_v[pl.ds(off, 16)]
                dst16 = dst_v[pl.ds(off, 16)]
                gid_v[pl.ds(off, 16)] = src16 + row_off
                for hh in range(2):
                    s = plsc.load_gather(att_v, [src16 * 4 + hh])
                    d = plsc.load_gather(att_v, [dst16 * 4 + (2 + hh)])
                    x = s + d
                    ex = jnp.exp(jnp.maximum(x, x * 0.2))
                    plsc.store_scatter(
                        ex_v, [off + lanes, jnp.full((16,), hh, jnp.int32)],
                        ex)
                return carry

            lax.fori_loop(0, CHUNK // 16, logits, 0)
            pltpu.async_copy(hc_hbm.at[gid_v], rows_v, sem)

        def finish_chunk(b):
            src_v, dst_v, gid_v, rows_v, ex_v, sem = b
            pltpu.make_async_copy(hc_hbm.at[gid_v], rows_v, sem).wait()

            def scale(g, carry):
                off = g * 16
                ex0 = plsc.load_gather(
                    ex_v, [off + lanes, jnp.zeros((16,), jnp.int32)])
                ex1 = plsc.load_gather(
                    ex_v, [off + lanes, jnp.full((16,), 1, jnp.int32)])
                for e in range(16):
                    row = off + e
                    for hh, exv in ((0, ex0), (1, ex1)):
                        vsc = jnp.full((16,), exv[e])
                        for j in range(2):
                            col = hh * D + j * 16
                            rows_v[row, pl.ds(col, 16)] = (
                                rows_v[row, pl.ds(col, 16)] * vsc)
                return carry

            lax.fori_loop(0, CHUNK // 16, scale, 0)
            pltpu.sync_copy(rows_v, msg_sh.at[dst_v], add=True)
            pltpu.sync_copy(ex_v, den_sh.at[dst_v], add=True)

        n_steps = n_chunks // 2
        start_chunk(0, bufs[0])
        start_chunk(1, bufs[1])

        def step(i, carry):
            finish_chunk(bufs[0])

            @pl.when(i < n_steps - 1)
            def _():
                start_chunk(2 * i + 2, bufs[0])

            finish_chunk(bufs[1])

            @pl.when(i < n_steps - 1)
            def _():
                start_chunk(2 * i + 3, bufs[1])

            return carry

        lax.fori_loop(0, n_steps, step, 0)
        plsc.subcore_barrier()

        pltpu.sync_copy(msg_sh.at[pl.ds(sid * rpt, rpt)], out_msg.at[cid, sid])
        pltpu.sync_copy(den_sh.at[pl.ds(sid * rpt, rpt)], out_den.at[cid, sid])

    return kern


def _finish_body(msg_ref, den_ref, feat_ref, out_ref):
    parts = []
    for c in range(NC):
        for hh in range(2):
            m = msg_ref[c][:, hh * D:(hh + 1) * D]
            dn = den_ref[c][:, hh:hh + 1]
            parts.append(m / (dn + 1e-9))
    out_ref[...] = feat_ref[...] + jnp.concatenate(parts, axis=1)


def _finish(msg, den, feat, block_rows):
    n = feat.shape[0]
    grid = n // block_rows
    return pl.pallas_call(
        _finish_body,
        grid=(grid,),
        in_specs=[
            pl.BlockSpec((NC, block_rows, HD2), lambda i: (0, i, 0)),
            pl.BlockSpec((NC, block_rows, 8), lambda i: (0, i, 0)),
            pl.BlockSpec((block_rows, H * D), lambda i: (i, 0)),
        ],
        out_specs=pl.BlockSpec((block_rows, H * D), lambda i: (i, 0)),
        out_shape=jax.ShapeDtypeStruct((n, H * D), jnp.float32),
    )(msg, den, feat)


def kernel(feat, edge_index, W_content, W_pos, attn_src, attn_dst,
           pos_attn_src, pos_attn_dst, att_comb):
    n, in_dim = feat.shape
    e = edge_index.shape[1]
    content_dim = in_dim - POS_DIM

    c0 = att_comb[:, 0]
    c1 = att_comb[:, 1]
    wc3 = W_content.reshape(content_dim, H, D)
    wp3 = W_pos.reshape(POS_DIM, H, D // 4)
    a_src_c = jnp.einsum("chd,hd->ch", wc3, attn_src[0]) * c0
    a_dst_c = jnp.einsum("chd,hd->ch", wc3, attn_dst[0]) * c0
    a_src_p = jnp.einsum("phk,hk->ph", wp3, pos_attn_src[0]) * c1
    a_dst_p = jnp.einsum("phk,hk->ph", wp3, pos_attn_dst[0]) * c1
    W_a = jnp.concatenate(
        [jnp.concatenate([a_src_c, a_src_p], axis=0),
         jnp.concatenate([a_dst_c, a_dst_p], axis=0)], axis=1)
    W_h = jnp.concatenate(
        [W_content, jnp.zeros((POS_DIM, H * D), jnp.float32)], axis=0)

    hc2, attn8 = _node_proj(feat, W_h, W_a, block_rows=1000)

    att_tabs = []
    for c in range(NC):
        att_tabs.append(jnp.concatenate(
            [attn8[:, 2 * c:2 * c + 2], attn8[:, 4 + 2 * c:4 + 2 * c + 2]],
            axis=1).reshape(-1))

    zmsg = jnp.zeros((n // NS, HD2), jnp.float32)
    zden = jnp.zeros((n // NS, 8), jnp.float32)
    out_msg, out_den = _edge_kernel(n, e)(
        att_tabs[0], att_tabs[1], edge_index[0], edge_index[1],
        hc2.reshape(NC * n, HD2), zmsg, zden)

    return _finish(out_msg.reshape(NC, n, HD2),
                   out_den.reshape(NC, n, 8), feat, block_rows=1000)
